# Initial kernel scaffold; baseline (speedup 1.0000x reference)
import functools

import jax
import jax.numpy as jnp
from jax import lax
from jax.experimental import pallas as pl
from jax.experimental.pallas import tpu as pltpu
from jax.experimental.pallas import tpu_sc as plsc

H = 768
NC = 2
NS = 16
NW = NC * NS
LANES = 16


def _sc_gather_body(ids_hbm, x0_hbm, y1_hbm, x2_hbm, y3_hbm,
                    word_tab, x_tab, y_tab, h_tab, w_tab,
                    spatial_out, words_out,
                    idx_v, gbuf, sem, *, n_tokens, t_chunk):
    tpw = n_tokens // NW
    nch = tpw // t_chunk
    wid = lax.axis_index("s") * NC + lax.axis_index("c")
    base = wid * tpw

    pltpu.sync_copy(x0_hbm.at[pl.ds(base, tpw)], idx_v.at[0])
    pltpu.sync_copy(y1_hbm.at[pl.ds(base, tpw)], idx_v.at[1])
    pltpu.sync_copy(x2_hbm.at[pl.ds(base, tpw)], idx_v.at[2])
    pltpu.sync_copy(y3_hbm.at[pl.ds(base, tpw)], idx_v.at[3])
    pltpu.sync_copy(ids_hbm.at[pl.ds(base, tpw)], idx_v.at[6])

    @pl.loop(0, tpw // LANES)
    def _compute_hw(j):
        s = pl.ds(j * LANES, LANES)
        idx_v[4, s] = jnp.abs(idx_v[3, s] - idx_v[1, s])
        idx_v[5, s] = jnp.abs(idx_v[2, s] - idx_v[0, s])

    tabs = (x_tab, y_tab, x_tab, y_tab, h_tab, w_tab, word_tab)

    @pl.loop(0, nch)
    def _chunk(c):
        tok0 = c * t_chunk
        cps = [
            pltpu.async_copy(tabs[k].at[idx_v.at[k, pl.ds(tok0, t_chunk)]],
                             gbuf.at[k], sem)
            for k in range(7)
        ]
        for cp in cps:
            cp.wait()

        @pl.loop(0, (t_chunk * H) // LANES)
        def _sum(j):
            t = j // (H // LANES)
            s = pl.ds((j % (H // LANES)) * LANES, LANES)
            acc = gbuf[0, t, s] + gbuf[1, t, s]
            acc = acc + gbuf[2, t, s]
            acc = acc + gbuf[3, t, s]
            acc = acc + gbuf[4, t, s]
            gbuf[0, t, s] = acc + gbuf[5, t, s]

        dst = pl.ds(base + tok0, t_chunk)
        pltpu.sync_copy(gbuf.at[0], spatial_out.at[dst])
        pltpu.sync_copy(gbuf.at[6], words_out.at[dst])


def _make_sc_gather(n_tokens, t_chunk=16):
    body = functools.partial(_sc_gather_body, n_tokens=n_tokens,
                             t_chunk=t_chunk)
    return pl.kernel(
        body,
        out_type=(
            jax.ShapeDtypeStruct((n_tokens, H), jnp.float32),
            jax.ShapeDtypeStruct((n_tokens, H), jnp.float32),
        ),
        mesh=plsc.VectorSubcoreMesh(core_axis_name="c", subcore_axis_name="s"),
        scratch_types=[
            pltpu.VMEM((7, n_tokens // NW), jnp.int32),
            pltpu.VMEM((7, t_chunk, H), jnp.float32),
            pltpu.SemaphoreType.DMA,
        ],
    )


def _tc_body(spatial_ref, words_ref, pos_ref, tte_ref, w1_ref, b1_ref,
             w2_ref, b2_ref, g_ref, bb_ref, out_ref):
    x = spatial_ref[...]
    h = lax.dot_general(x.astype(jnp.bfloat16), w1_ref[...].astype(jnp.bfloat16),
                        (((1,), (1,)), ((), ())),
                        preferred_element_type=jnp.float32)
    h = jnp.maximum(h + b1_ref[...], 0.0)
    t = lax.dot_general(h.astype(jnp.bfloat16), w2_ref[...].astype(jnp.bfloat16),
                        (((1,), (1,)), ((), ())),
                        preferred_element_type=jnp.float32)
    e = words_ref[...] + pos_ref[...] + (t + b2_ref[...]) + tte_ref[0:1, :]
    mu = jnp.mean(e, axis=1, keepdims=True)
    ec = e - mu
    var = jnp.mean(ec * ec, axis=1, keepdims=True)
    out_ref[...] = ec * lax.rsqrt(var + 1e-12) * g_ref[...] + bb_ref[...]


def _make_tc_mlp(n_tokens, seq, tb=512):
    n_blocks = n_tokens // tb
    pos_blocks = seq // tb
    full = lambda i: (0, 0)
    return pl.pallas_call(
        _tc_body,
        grid=(n_blocks,),
        in_specs=[
            pl.BlockSpec((tb, H), lambda i: (i, 0)),
            pl.BlockSpec((tb, H), lambda i: (i, 0)),
            pl.BlockSpec((tb, H), lambda i: (i % pos_blocks, 0)),
            pl.BlockSpec((2, H), full),
            pl.BlockSpec((H, H), full),
            pl.BlockSpec((1, H), full),
            pl.BlockSpec((H, H), full),
            pl.BlockSpec((1, H), full),
            pl.BlockSpec((1, H), full),
            pl.BlockSpec((1, H), full),
        ],
        out_specs=pl.BlockSpec((tb, H), lambda i: (i, 0)),
        out_shape=jax.ShapeDtypeStruct((n_tokens, H), jnp.float32),
    )


def kernel(input_ids, bbox, word_emb, exp_pos_emb, x_emb, y_emb, h_emb,
           w_emb, tok_type_emb, W1, b1, W2, b2, ln_g, ln_b):
    b, s = input_ids.shape
    n = b * s
    ids = input_ids.reshape(n)
    x0 = bbox[:, :, 0].reshape(n)
    y1 = bbox[:, :, 1].reshape(n)
    x2 = bbox[:, :, 2].reshape(n)
    y3 = bbox[:, :, 3].reshape(n)

    sc = _make_sc_gather(n)
    spatial, words = sc(ids, x0, y1, x2, y3,
                        word_emb, x_emb, y_emb, h_emb, w_emb)

    tc = _make_tc_mlp(n, s)
    out = tc(spatial, words, exp_pos_emb, tok_type_emb,
             W1, b1.reshape(1, H), W2, b2.reshape(1, H),
             ln_g.reshape(1, H), ln_b.reshape(1, H))
    return out.reshape(b, s, H)


# speedup vs baseline: 1.7483x; 1.7483x over previous
import functools

import jax
import jax.numpy as jnp
from jax import lax
from jax.experimental import pallas as pl
from jax.experimental.pallas import tpu as pltpu
from jax.experimental.pallas import tpu_sc as plsc

H = 768
NC = 2
NS = 16
NW = NC * NS
LANES = 16


def _sc_gather_body(ids_hbm, x0_hbm, y1_hbm, x2_hbm, y3_hbm,
                    word_tab, x_tab, y_tab, h_tab, w_tab,
                    spatial_out, words_out,
                    i0, i1, i2, i3, i4, i5, i6, gbuf, sem,
                    *, n_tokens, t_chunk):
    tpw = n_tokens // NW
    nch = tpw // t_chunk
    wid = lax.axis_index("s") * NC + lax.axis_index("c")
    base = wid * tpw
    idx = (i0, i1, i2, i3, i4, i5, i6)

    pltpu.sync_copy(x0_hbm.at[pl.ds(base, tpw)], i0)
    pltpu.sync_copy(y1_hbm.at[pl.ds(base, tpw)], i1)
    pltpu.sync_copy(x2_hbm.at[pl.ds(base, tpw)], i2)
    pltpu.sync_copy(y3_hbm.at[pl.ds(base, tpw)], i3)
    pltpu.sync_copy(ids_hbm.at[pl.ds(base, tpw)], i6)

    @pl.loop(0, tpw // LANES)
    def _compute_hw(j):
        s = pl.ds(j * LANES, LANES)
        i4[s] = jnp.abs(i3[s] - i1[s])
        i5[s] = jnp.abs(i2[s] - i0[s])

    tabs = (x_tab, y_tab, x_tab, y_tab, h_tab, w_tab, word_tab)

    @pl.loop(0, nch)
    def _chunk(c):
        tok0 = c * t_chunk
        cps = [
            pltpu.async_copy(tabs[k].at[idx[k].at[pl.ds(tok0, t_chunk)]],
                             gbuf.at[k], sem)
            for k in range(7)
        ]
        for cp in cps:
            cp.wait()

        @pl.loop(0, (t_chunk * H) // LANES)
        def _sum(j):
            t = j // (H // LANES)
            s = pl.ds((j % (H // LANES)) * LANES, LANES)
            acc = gbuf[0, t, s] + gbuf[1, t, s]
            acc = acc + gbuf[2, t, s]
            acc = acc + gbuf[3, t, s]
            acc = acc + gbuf[4, t, s]
            gbuf[0, t, s] = acc + gbuf[5, t, s]

        dst = pl.ds(base + tok0, t_chunk)
        pltpu.sync_copy(gbuf.at[0], spatial_out.at[dst])
        pltpu.sync_copy(gbuf.at[6], words_out.at[dst])


def _make_sc_gather(n_tokens, t_chunk=16):
    body = functools.partial(_sc_gather_body, n_tokens=n_tokens,
                             t_chunk=t_chunk)
    return pl.kernel(
        body,
        out_type=(
            jax.ShapeDtypeStruct((n_tokens, H), jnp.float32),
            jax.ShapeDtypeStruct((n_tokens, H), jnp.float32),
        ),
        mesh=plsc.VectorSubcoreMesh(core_axis_name="c", subcore_axis_name="s"),
        scratch_types=(
            [pltpu.VMEM((n_tokens // NW,), jnp.int32) for _ in range(7)]
            + [pltpu.VMEM((7, t_chunk, H), jnp.float32),
               pltpu.SemaphoreType.DMA]
        ),
    )


def _tc_body(spatial_ref, words_ref, pos_ref, tte_ref, w1_ref, b1_ref,
             w2_ref, b2_ref, g_ref, bb_ref, out_ref):
    x = spatial_ref[...]
    h = lax.dot_general(x.astype(jnp.bfloat16), w1_ref[...].astype(jnp.bfloat16),
                        (((1,), (1,)), ((), ())),
                        preferred_element_type=jnp.float32)
    h = jnp.maximum(h + b1_ref[...], 0.0)
    t = lax.dot_general(h.astype(jnp.bfloat16), w2_ref[...].astype(jnp.bfloat16),
                        (((1,), (1,)), ((), ())),
                        preferred_element_type=jnp.float32)
    e = words_ref[...] + pos_ref[...] + (t + b2_ref[...]) + tte_ref[0:1, :]
    mu = jnp.mean(e, axis=1, keepdims=True)
    ec = e - mu
    var = jnp.mean(ec * ec, axis=1, keepdims=True)
    out_ref[...] = ec * lax.rsqrt(var + 1e-12) * g_ref[...] + bb_ref[...]


def _make_tc_mlp(n_tokens, seq, tb=512):
    n_blocks = n_tokens // tb
    pos_blocks = seq // tb
    full = lambda i: (0, 0)
    return pl.pallas_call(
        _tc_body,
        grid=(n_blocks,),
        in_specs=[
            pl.BlockSpec((tb, H), lambda i: (i, 0)),
            pl.BlockSpec((tb, H), lambda i: (i, 0)),
            pl.BlockSpec((tb, H), lambda i: (i % pos_blocks, 0)),
            pl.BlockSpec((2, H), full),
            pl.BlockSpec((H, H), full),
            pl.BlockSpec((1, H), full),
            pl.BlockSpec((H, H), full),
            pl.BlockSpec((1, H), full),
            pl.BlockSpec((1, H), full),
            pl.BlockSpec((1, H), full),
        ],
        out_specs=pl.BlockSpec((tb, H), lambda i: (i, 0)),
        out_shape=jax.ShapeDtypeStruct((n_tokens, H), jnp.float32),
    )


def kernel(input_ids, bbox, word_emb, exp_pos_emb, x_emb, y_emb, h_emb,
           w_emb, tok_type_emb, W1, b1, W2, b2, ln_g, ln_b):
    b, s = input_ids.shape
    n = b * s
    ids = input_ids.reshape(n)
    x0 = bbox[:, :, 0].reshape(n)
    y1 = bbox[:, :, 1].reshape(n)
    x2 = bbox[:, :, 2].reshape(n)
    y3 = bbox[:, :, 3].reshape(n)

    sc = _make_sc_gather(n)
    spatial, words = sc(ids, x0, y1, x2, y3,
                        word_emb, x_emb, y_emb, h_emb, w_emb)

    tc = _make_tc_mlp(n, s)
    out = tc(spatial, words, exp_pos_emb, tok_type_emb,
             W1, b1.reshape(1, H), W2, b2.reshape(1, H),
             ln_g.reshape(1, H), ln_b.reshape(1, H))
    return out.reshape(b, s, H)


# speedup vs baseline: 2.2871x; 1.3082x over previous
import functools

import jax
import jax.numpy as jnp
from jax import lax
from jax.experimental import pallas as pl
from jax.experimental.pallas import tpu as pltpu
from jax.experimental.pallas import tpu_sc as plsc

H = 768
NC = 2
NS = 16
NW = NC * NS
LANES = 16


def _sc_gather_body(ids_hbm, x0_hbm, y1_hbm, x2_hbm, y3_hbm,
                    word_tab, x_tab, y_tab, h_tab, w_tab,
                    spatial_out, words_out,
                    i0, i1, i2, i3, i4, i5, i6, gbuf,
                    gsem0, gsem1, ssem0, ssem1,
                    *, n_tokens, t_chunk):
    tpw = n_tokens // NW
    nch = tpw // t_chunk
    wid = lax.axis_index("s") * NC + lax.axis_index("c")
    base = wid * tpw
    idx = (i0, i1, i2, i3, i4, i5, i6)
    gsems = (gsem0, gsem1)
    ssems = (ssem0, ssem1)

    pltpu.sync_copy(x0_hbm.at[pl.ds(base, tpw)], i0)
    pltpu.sync_copy(y1_hbm.at[pl.ds(base, tpw)], i1)
    pltpu.sync_copy(x2_hbm.at[pl.ds(base, tpw)], i2)
    pltpu.sync_copy(y3_hbm.at[pl.ds(base, tpw)], i3)
    pltpu.sync_copy(ids_hbm.at[pl.ds(base, tpw)], i6)

    @pl.loop(0, tpw // LANES)
    def _compute_hw(j):
        s = pl.ds(j * LANES, LANES)
        i4[s] = jnp.abs(i3[s] - i1[s])
        i5[s] = jnp.abs(i2[s] - i0[s])

    tabs = (x_tab, y_tab, x_tab, y_tab, h_tab, w_tab, word_tab)

    def fire_gathers(c, p):
        tok0 = c * t_chunk
        for k in range(7):
            pltpu.async_copy(tabs[k].at[idx[k].at[pl.ds(tok0, t_chunk)]],
                             gbuf.at[p, k], gsems[p])

    def wait_gathers(p):
        for k in range(7):
            pltpu.make_async_copy(
                tabs[k].at[idx[k].at[pl.ds(0, t_chunk)]],
                gbuf.at[p, k], gsems[p]).wait()

    fire_gathers(0, 0)
    fire_gathers(1, 1)

    @pl.loop(0, nch // 2)
    def _super(sc):
        for p in range(2):
            c = sc * 2 + p
            wait_gathers(p)

            @pl.loop(0, (t_chunk * H) // LANES)
            def _sum(j):
                t = j // (H // LANES)
                s = pl.ds((j % (H // LANES)) * LANES, LANES)
                acc = gbuf[p, 0, t, s] + gbuf[p, 1, t, s]
                acc = acc + gbuf[p, 2, t, s]
                acc = acc + gbuf[p, 3, t, s]
                acc = acc + gbuf[p, 4, t, s]
                gbuf[p, 0, t, s] = acc + gbuf[p, 5, t, s]

            dst = pl.ds(base + c * t_chunk, t_chunk)
            st0 = pltpu.async_copy(gbuf.at[p, 0], spatial_out.at[dst], ssems[p])
            st1 = pltpu.async_copy(gbuf.at[p, 6], words_out.at[dst], ssems[p])
            st0.wait()
            st1.wait()

            @pl.when(c + 2 < nch)
            def _():
                fire_gathers(c + 2, p)


def _make_sc_gather(n_tokens, t_chunk=8):
    body = functools.partial(_sc_gather_body, n_tokens=n_tokens,
                             t_chunk=t_chunk)
    return pl.kernel(
        body,
        out_type=(
            jax.ShapeDtypeStruct((n_tokens, H), jnp.float32),
            jax.ShapeDtypeStruct((n_tokens, H), jnp.float32),
        ),
        mesh=plsc.VectorSubcoreMesh(core_axis_name="c", subcore_axis_name="s"),
        scratch_types=(
            [pltpu.VMEM((n_tokens // NW,), jnp.int32) for _ in range(7)]
            + [pltpu.VMEM((2, 7, t_chunk, H), jnp.float32),
               pltpu.SemaphoreType.DMA, pltpu.SemaphoreType.DMA,
               pltpu.SemaphoreType.DMA, pltpu.SemaphoreType.DMA]
        ),
    )


def _tc_body(spatial_ref, words_ref, pos_ref, tte_ref, w1_ref, b1_ref,
             w2_ref, b2_ref, g_ref, bb_ref, out_ref):
    x = spatial_ref[...]
    h = lax.dot_general(x.astype(jnp.bfloat16), w1_ref[...].astype(jnp.bfloat16),
                        (((1,), (1,)), ((), ())),
                        preferred_element_type=jnp.float32)
    h = jnp.maximum(h + b1_ref[...], 0.0)
    t = lax.dot_general(h.astype(jnp.bfloat16), w2_ref[...].astype(jnp.bfloat16),
                        (((1,), (1,)), ((), ())),
                        preferred_element_type=jnp.float32)
    e = words_ref[...] + pos_ref[...] + (t + b2_ref[...]) + tte_ref[0:1, :]
    mu = jnp.mean(e, axis=1, keepdims=True)
    ec = e - mu
    var = jnp.mean(ec * ec, axis=1, keepdims=True)
    out_ref[...] = ec * lax.rsqrt(var + 1e-12) * g_ref[...] + bb_ref[...]


def _make_tc_mlp(n_tokens, seq, tb=512):
    n_blocks = n_tokens // tb
    pos_blocks = seq // tb
    full = lambda i: (0, 0)
    return pl.pallas_call(
        _tc_body,
        grid=(n_blocks,),
        in_specs=[
            pl.BlockSpec((tb, H), lambda i: (i, 0)),
            pl.BlockSpec((tb, H), lambda i: (i, 0)),
            pl.BlockSpec((tb, H), lambda i: (i % pos_blocks, 0)),
            pl.BlockSpec((2, H), full),
            pl.BlockSpec((H, H), full),
            pl.BlockSpec((1, H), full),
            pl.BlockSpec((H, H), full),
            pl.BlockSpec((1, H), full),
            pl.BlockSpec((1, H), full),
            pl.BlockSpec((1, H), full),
        ],
        out_specs=pl.BlockSpec((tb, H), lambda i: (i, 0)),
        out_shape=jax.ShapeDtypeStruct((n_tokens, H), jnp.float32),
    )


def kernel(input_ids, bbox, word_emb, exp_pos_emb, x_emb, y_emb, h_emb,
           w_emb, tok_type_emb, W1, b1, W2, b2, ln_g, ln_b):
    b, s = input_ids.shape
    n = b * s
    ids = input_ids.reshape(n)
    x0 = bbox[:, :, 0].reshape(n)
    y1 = bbox[:, :, 1].reshape(n)
    x2 = bbox[:, :, 2].reshape(n)
    y3 = bbox[:, :, 3].reshape(n)

    sc = _make_sc_gather(n)
    spatial, words = sc(ids, x0, y1, x2, y3,
                        word_emb, x_emb, y_emb, h_emb, w_emb)

    tc = _make_tc_mlp(n, s)
    out = tc(spatial, words, exp_pos_emb, tok_type_emb,
             W1, b1.reshape(1, H), W2, b2.reshape(1, H),
             ln_g.reshape(1, H), ln_b.reshape(1, H))
    return out.reshape(b, s, H)


# speedup vs baseline: 2.8999x; 1.2679x over previous
import functools

import jax
import jax.numpy as jnp
from jax import lax
from jax.experimental import pallas as pl
from jax.experimental.pallas import tpu as pltpu
from jax.experimental.pallas import tpu_sc as plsc

H = 768
NC = 2
NS = 16
NW = NC * NS
POS2D = 1024


def _sc_words_body(ids_hbm, word_tab, words_out,
                   idx, gbuf, gsem0, gsem1, ssem0, ssem1,
                   *, n_tokens, t_chunk):
    tpw = n_tokens // NW
    nch = tpw // t_chunk
    wid = lax.axis_index("s") * NC + lax.axis_index("c")
    base = wid * tpw
    gsems = (gsem0, gsem1)
    ssems = (ssem0, ssem1)

    pltpu.sync_copy(ids_hbm.at[pl.ds(base, tpw)], idx)

    def fire(c, p):
        pltpu.async_copy(
            word_tab.at[idx.at[pl.ds(c * t_chunk, t_chunk)]],
            gbuf.at[p], gsems[p])

    def wait(p):
        pltpu.make_async_copy(
            word_tab.at[idx.at[pl.ds(0, t_chunk)]],
            gbuf.at[p], gsems[p]).wait()

    fire(0, 0)
    fire(1, 1)

    @pl.loop(0, nch // 2)
    def _super(sc):
        for p in range(2):
            c = sc * 2 + p
            wait(p)
            dst = pl.ds(base + c * t_chunk, t_chunk)
            st = pltpu.async_copy(gbuf.at[p], words_out.at[dst], ssems[p])
            st.wait()

            @pl.when(c + 2 < nch)
            def _():
                fire(c + 2, p)


def _make_sc_words(n_tokens, t_chunk=16):
    body = functools.partial(_sc_words_body, n_tokens=n_tokens,
                             t_chunk=t_chunk)
    return pl.kernel(
        body,
        out_type=jax.ShapeDtypeStruct((n_tokens, H), jnp.float32),
        mesh=plsc.VectorSubcoreMesh(core_axis_name="c", subcore_axis_name="s"),
        scratch_types=(
            pltpu.VMEM((n_tokens // NW,), jnp.int32),
            pltpu.VMEM((2, t_chunk, H), jnp.float32),
            pltpu.SemaphoreType.DMA, pltpu.SemaphoreType.DMA,
            pltpu.SemaphoreType.DMA, pltpu.SemaphoreType.DMA,
        ),
    )


def _tc_body(words_ref, x0_ref, y1_ref, x2_ref, y3_ref,
             xt_ref, yt_ref, ht_ref, wt_ref,
             pos_ref, tte_ref, w1_ref, b1_ref, w2_ref, b2_ref,
             g_ref, bb_ref, out_ref, *, tb):
    x0 = x0_ref[0]
    y1 = y1_ref[0]
    x2 = x2_ref[0]
    y3 = y3_ref[0]
    hh = jnp.abs(y3 - y1)
    ww = jnp.abs(x2 - x0)

    rows = lax.broadcasted_iota(jnp.int32, (POS2D, tb), 0)

    def onehot_t(v):
        return (v == rows).astype(jnp.bfloat16)

    dn = (((0,), (0,)), ((), ()))
    spatial = lax.dot_general(onehot_t(x0) + onehot_t(x2), xt_ref[...], dn,
                              preferred_element_type=jnp.float32)
    spatial += lax.dot_general(onehot_t(y1) + onehot_t(y3), yt_ref[...], dn,
                               preferred_element_type=jnp.float32)
    spatial += lax.dot_general(onehot_t(hh), ht_ref[...], dn,
                               preferred_element_type=jnp.float32)
    spatial += lax.dot_general(onehot_t(ww), wt_ref[...], dn,
                               preferred_element_type=jnp.float32)

    h = lax.dot_general(spatial.astype(jnp.bfloat16), w1_ref[...],
                        (((1,), (1,)), ((), ())),
                        preferred_element_type=jnp.float32)
    h = jnp.maximum(h + b1_ref[...], 0.0)
    t = lax.dot_general(h.astype(jnp.bfloat16), w2_ref[...],
                        (((1,), (1,)), ((), ())),
                        preferred_element_type=jnp.float32)
    e = words_ref[...] + pos_ref[...] + (t + b2_ref[...]) + tte_ref[0:1, :]
    mu = jnp.mean(e, axis=1, keepdims=True)
    ec = e - mu
    var = jnp.mean(ec * ec, axis=1, keepdims=True)
    out_ref[...] = ec * lax.rsqrt(var + 1e-12) * g_ref[...] + bb_ref[...]


def _make_tc(n_tokens, seq, tb=1024):
    n_blocks = n_tokens // tb
    pos_blocks = seq // tb
    per_pos = n_blocks // pos_blocks
    tmap = lambda i: lax.rem(i, per_pos) * pos_blocks + lax.div(i, per_pos)
    full = lambda i: (0, 0)
    body = functools.partial(_tc_body, tb=tb)
    return pl.pallas_call(
        body,
        grid=(n_blocks,),
        in_specs=[
            pl.BlockSpec((tb, H), lambda i: (tmap(i), 0)),
            pl.BlockSpec((1, 1, tb), lambda i: (tmap(i), 0, 0)),
            pl.BlockSpec((1, 1, tb), lambda i: (tmap(i), 0, 0)),
            pl.BlockSpec((1, 1, tb), lambda i: (tmap(i), 0, 0)),
            pl.BlockSpec((1, 1, tb), lambda i: (tmap(i), 0, 0)),
            pl.BlockSpec((POS2D, H), full),
            pl.BlockSpec((POS2D, H), full),
            pl.BlockSpec((POS2D, H), full),
            pl.BlockSpec((POS2D, H), full),
            pl.BlockSpec((tb, H), lambda i: (lax.div(i, per_pos), 0)),
            pl.BlockSpec((2, H), full),
            pl.BlockSpec((H, H), full),
            pl.BlockSpec((1, H), full),
            pl.BlockSpec((H, H), full),
            pl.BlockSpec((1, H), full),
            pl.BlockSpec((1, H), full),
            pl.BlockSpec((1, H), full),
        ],
        out_specs=pl.BlockSpec((tb, H), lambda i: (tmap(i), 0)),
        out_shape=jax.ShapeDtypeStruct((n_tokens, H), jnp.float32),
    )


def kernel(input_ids, bbox, word_emb, exp_pos_emb, x_emb, y_emb, h_emb,
           w_emb, tok_type_emb, W1, b1, W2, b2, ln_g, ln_b):
    b, s = input_ids.shape
    n = b * s
    tb = 1024
    ids = input_ids.reshape(n)
    x0 = bbox[:, :, 0].reshape(n // tb, 1, tb)
    y1 = bbox[:, :, 1].reshape(n // tb, 1, tb)
    x2 = bbox[:, :, 2].reshape(n // tb, 1, tb)
    y3 = bbox[:, :, 3].reshape(n // tb, 1, tb)

    sc = _make_sc_words(n)
    words = sc(ids, word_emb)

    tc = _make_tc(n, s, tb)
    out = tc(words, x0, y1, x2, y3,
             x_emb.astype(jnp.bfloat16), y_emb.astype(jnp.bfloat16),
             h_emb.astype(jnp.bfloat16), w_emb.astype(jnp.bfloat16),
             exp_pos_emb, tok_type_emb,
             W1.astype(jnp.bfloat16), b1.reshape(1, H),
             W2.astype(jnp.bfloat16), b2.reshape(1, H),
             ln_g.reshape(1, H), ln_b.reshape(1, H))
    return out.reshape(b, s, H)


# speedup vs baseline: 3.4841x; 1.2015x over previous
import functools

import jax
import jax.numpy as jnp
from jax import lax
from jax.experimental import pallas as pl
from jax.experimental.pallas import tpu as pltpu
from jax.experimental.pallas import tpu_sc as plsc

H = 768
NC = 2
NS = 16
NW = NC * NS
POS2D = 1024


def _sc_words_body(ids_hbm, word_tab, words_out,
                   idx, gbuf, gsem0, gsem1, ssem0, ssem1,
                   *, n_tokens, t_chunk):
    tpw = n_tokens // NW
    nch = tpw // t_chunk
    wid = lax.axis_index("s") * NC + lax.axis_index("c")
    base = wid * tpw
    gsems = (gsem0, gsem1)
    ssems = (ssem0, ssem1)

    pltpu.sync_copy(ids_hbm.at[pl.ds(base, tpw)], idx)

    def fire(c, p):
        pltpu.async_copy(
            word_tab.at[idx.at[pl.ds(c * t_chunk, t_chunk)]],
            gbuf.at[p], gsems[p])

    def wait(p):
        pltpu.make_async_copy(
            word_tab.at[idx.at[pl.ds(0, t_chunk)]],
            gbuf.at[p], gsems[p]).wait()

    fire(0, 0)
    fire(1, 1)

    @pl.loop(0, nch // 2)
    def _super(sc):
        for p in range(2):
            c = sc * 2 + p
            wait(p)
            dst = pl.ds(base + c * t_chunk, t_chunk)
            st = pltpu.async_copy(gbuf.at[p], words_out.at[dst], ssems[p])
            st.wait()

            @pl.when(c + 2 < nch)
            def _():
                fire(c + 2, p)


def _make_sc_words(n_tokens, t_chunk=16):
    body = functools.partial(_sc_words_body, n_tokens=n_tokens,
                             t_chunk=t_chunk)
    return pl.kernel(
        body,
        out_type=jax.ShapeDtypeStruct((n_tokens, H), jnp.float32),
        mesh=plsc.VectorSubcoreMesh(core_axis_name="c", subcore_axis_name="s"),
        scratch_types=(
            pltpu.VMEM((n_tokens // NW,), jnp.int32),
            pltpu.VMEM((2, t_chunk, H), jnp.float32),
            pltpu.SemaphoreType.DMA, pltpu.SemaphoreType.DMA,
            pltpu.SemaphoreType.DMA, pltpu.SemaphoreType.DMA,
        ),
    )


def _tc_body(words_ref, x0_ref, y1_ref, x2_ref, y3_ref,
             xt_ref, yt_ref, ht_ref, wt_ref, sc_ref,
             pos_ref, tte_ref, w1_ref, b1_ref, w2_ref, b2_ref,
             g_ref, bb_ref, out_ref, *, tb):
    x0 = x0_ref[0]
    y1 = y1_ref[0]
    x2 = x2_ref[0]
    y3 = y3_ref[0]
    hh = jnp.abs(y3 - y1)
    ww = jnp.abs(x2 - x0)

    rows = lax.broadcasted_iota(jnp.int32, (POS2D, tb), 0)

    def onehot_t(v):
        return (v == rows).astype(jnp.int32)

    dn = (((0,), (0,)), ((), ()))
    acc = lax.dot_general((onehot_t(x0) + onehot_t(x2)).astype(jnp.int8),
                          xt_ref[...], dn,
                          preferred_element_type=jnp.int32)
    acc += lax.dot_general((onehot_t(y1) + onehot_t(y3)).astype(jnp.int8),
                           yt_ref[...], dn,
                           preferred_element_type=jnp.int32)
    acc += lax.dot_general((hh == rows).astype(jnp.int8), ht_ref[...], dn,
                           preferred_element_type=jnp.int32)
    acc += lax.dot_general((ww == rows).astype(jnp.int8), wt_ref[...], dn,
                           preferred_element_type=jnp.int32)
    spatial = acc.astype(jnp.float32) * sc_ref[0:1, 0:1]

    h = lax.dot_general(spatial.astype(jnp.bfloat16), w1_ref[...],
                        (((1,), (1,)), ((), ())),
                        preferred_element_type=jnp.float32)
    h = jnp.maximum(h + b1_ref[...], 0.0)
    t = lax.dot_general(h.astype(jnp.bfloat16), w2_ref[...],
                        (((1,), (1,)), ((), ())),
                        preferred_element_type=jnp.float32)
    e = words_ref[...] + pos_ref[...] + (t + b2_ref[...]) + tte_ref[0:1, :]
    mu = jnp.mean(e, axis=1, keepdims=True)
    ec = e - mu
    var = jnp.mean(ec * ec, axis=1, keepdims=True)
    out_ref[...] = ec * lax.rsqrt(var + 1e-12) * g_ref[...] + bb_ref[...]


def _make_tc(n_tokens, seq, tb=1024):
    n_blocks = n_tokens // tb
    pos_blocks = seq // tb
    per_pos = n_blocks // pos_blocks
    tmap = lambda i: lax.rem(i, per_pos) * pos_blocks + lax.div(i, per_pos)
    full = lambda i: (0, 0)
    body = functools.partial(_tc_body, tb=tb)
    return pl.pallas_call(
        body,
        grid=(n_blocks,),
        in_specs=[
            pl.BlockSpec((tb, H), lambda i: (tmap(i), 0)),
            pl.BlockSpec((1, 1, tb), lambda i: (tmap(i), 0, 0)),
            pl.BlockSpec((1, 1, tb), lambda i: (tmap(i), 0, 0)),
            pl.BlockSpec((1, 1, tb), lambda i: (tmap(i), 0, 0)),
            pl.BlockSpec((1, 1, tb), lambda i: (tmap(i), 0, 0)),
            pl.BlockSpec((POS2D, H), full),
            pl.BlockSpec((POS2D, H), full),
            pl.BlockSpec((POS2D, H), full),
            pl.BlockSpec((POS2D, H), full),
            pl.BlockSpec((1, 1), full),
            pl.BlockSpec((tb, H), lambda i: (lax.div(i, per_pos), 0)),
            pl.BlockSpec((2, H), full),
            pl.BlockSpec((H, H), full),
            pl.BlockSpec((1, H), full),
            pl.BlockSpec((H, H), full),
            pl.BlockSpec((1, H), full),
            pl.BlockSpec((1, H), full),
            pl.BlockSpec((1, H), full),
        ],
        out_specs=pl.BlockSpec((tb, H), lambda i: (tmap(i), 0)),
        out_shape=jax.ShapeDtypeStruct((n_tokens, H), jnp.float32),
    )


def kernel(input_ids, bbox, word_emb, exp_pos_emb, x_emb, y_emb, h_emb,
           w_emb, tok_type_emb, W1, b1, W2, b2, ln_g, ln_b):
    b, s = input_ids.shape
    n = b * s
    tb = 1024
    ids = input_ids.reshape(n)
    x0 = bbox[:, :, 0].reshape(n // tb, 1, tb)
    y1 = bbox[:, :, 1].reshape(n // tb, 1, tb)
    x2 = bbox[:, :, 2].reshape(n // tb, 1, tb)
    y3 = bbox[:, :, 3].reshape(n // tb, 1, tb)

    sc = _make_sc_words(n)
    words = sc(ids, word_emb)

    absmax = jnp.maximum(
        jnp.maximum(jnp.max(jnp.abs(x_emb)), jnp.max(jnp.abs(y_emb))),
        jnp.maximum(jnp.max(jnp.abs(h_emb)), jnp.max(jnp.abs(w_emb))))
    scale = jnp.where(absmax > 0, absmax / 127.0, 1.0)
    quant = lambda t: jnp.round(t / scale).astype(jnp.int8)

    tc = _make_tc(n, s, tb)
    out = tc(words, x0, y1, x2, y3,
             quant(x_emb), quant(y_emb), quant(h_emb), quant(w_emb),
             scale.reshape(1, 1), exp_pos_emb, tok_type_emb,
             W1.astype(jnp.bfloat16), b1.reshape(1, H),
             W2.astype(jnp.bfloat16), b2.reshape(1, H),
             ln_g.reshape(1, H), ln_b.reshape(1, H))
    return out.reshape(b, s, H)


# speedup vs baseline: 3.4873x; 1.0009x over previous
import functools

import jax
import jax.numpy as jnp
from jax import lax
from jax.experimental import pallas as pl
from jax.experimental.pallas import tpu as pltpu
from jax.experimental.pallas import tpu_sc as plsc

H = 768
NC = 2
NS = 16
NW = NC * NS
POS2D = 1024


def _sc_words_body(ids_hbm, word_tab, words_out,
                   idx, gbuf, gsem0, gsem1, ssem0, ssem1,
                   *, n_tokens, t_chunk):
    tpw = n_tokens // NW
    nch = tpw // t_chunk
    wid = lax.axis_index("s") * NC + lax.axis_index("c")
    base = wid * tpw
    gsems = (gsem0, gsem1)
    ssems = (ssem0, ssem1)

    pltpu.sync_copy(ids_hbm.at[pl.ds(base, tpw)], idx)

    def fire(c, p):
        pltpu.async_copy(
            word_tab.at[idx.at[pl.ds(c * t_chunk, t_chunk)]],
            gbuf.at[p], gsems[p])

    def wait(p):
        pltpu.make_async_copy(
            word_tab.at[idx.at[pl.ds(0, t_chunk)]],
            gbuf.at[p], gsems[p]).wait()

    fire(0, 0)
    fire(1, 1)

    @pl.loop(0, nch // 2)
    def _super(sc):
        for p in range(2):
            c = sc * 2 + p
            wait(p)
            dst = pl.ds(base + c * t_chunk, t_chunk)
            st = pltpu.async_copy(gbuf.at[p], words_out.at[dst], ssems[p])
            st.wait()

            @pl.when(c + 2 < nch)
            def _():
                fire(c + 2, p)


def _make_sc_words(n_tokens, t_chunk=16):
    body = functools.partial(_sc_words_body, n_tokens=n_tokens,
                             t_chunk=t_chunk)
    return pl.kernel(
        body,
        out_type=jax.ShapeDtypeStruct((n_tokens, H), jnp.float32),
        mesh=plsc.VectorSubcoreMesh(core_axis_name="c", subcore_axis_name="s"),
        scratch_types=(
            pltpu.VMEM((n_tokens // NW,), jnp.int32),
            pltpu.VMEM((2, t_chunk, H), jnp.float32),
            pltpu.SemaphoreType.DMA, pltpu.SemaphoreType.DMA,
            pltpu.SemaphoreType.DMA, pltpu.SemaphoreType.DMA,
        ),
    )


def _tc_body(words_ref, x0_ref, y1_ref, x2_ref, y3_ref,
             xt_ref, yt_ref, ht_ref, wt_ref, sc_ref,
             pos_ref, tte_ref, w1_ref, b1_ref, w2_ref, b2_ref,
             g_ref, bb_ref, out_ref, *, tb):
    x0 = x0_ref[0]
    y1 = y1_ref[0]
    x2 = x2_ref[0]
    y3 = y3_ref[0]
    hh = jnp.abs(y3 - y1)
    ww = jnp.abs(x2 - x0)

    rows = lax.broadcasted_iota(jnp.int16, (POS2D, tb), 0)

    def onehot_t(v):
        return (v.astype(jnp.int16) == rows).astype(jnp.int16)

    dn = (((0,), (0,)), ((), ()))
    acc = lax.dot_general((onehot_t(x0) + onehot_t(x2)).astype(jnp.int8),
                          xt_ref[...], dn,
                          preferred_element_type=jnp.int32)
    acc += lax.dot_general((onehot_t(y1) + onehot_t(y3)).astype(jnp.int8),
                           yt_ref[...], dn,
                           preferred_element_type=jnp.int32)
    acc += lax.dot_general(onehot_t(hh).astype(jnp.int8), ht_ref[...], dn,
                           preferred_element_type=jnp.int32)
    acc += lax.dot_general(onehot_t(ww).astype(jnp.int8), wt_ref[...], dn,
                           preferred_element_type=jnp.int32)
    spatial = acc.astype(jnp.float32) * sc_ref[0:1, 0:1]

    h = lax.dot_general(spatial.astype(jnp.bfloat16), w1_ref[...],
                        (((1,), (1,)), ((), ())),
                        preferred_element_type=jnp.float32)
    h = jnp.maximum(h + b1_ref[...], 0.0)
    t = lax.dot_general(h.astype(jnp.bfloat16), w2_ref[...],
                        (((1,), (1,)), ((), ())),
                        preferred_element_type=jnp.float32)
    e = words_ref[...] + pos_ref[...] + (t + b2_ref[...]) + tte_ref[0:1, :]
    mu = jnp.mean(e, axis=1, keepdims=True)
    ec = e - mu
    var = jnp.mean(ec * ec, axis=1, keepdims=True)
    out_ref[...] = ec * lax.rsqrt(var + 1e-12) * g_ref[...] + bb_ref[...]


def _make_tc(n_tokens, seq, tb=1024):
    n_blocks = n_tokens // tb
    pos_blocks = seq // tb
    per_pos = n_blocks // pos_blocks
    tmap = lambda i: lax.rem(i, per_pos) * pos_blocks + lax.div(i, per_pos)
    full = lambda i: (0, 0)
    body = functools.partial(_tc_body, tb=tb)
    return pl.pallas_call(
        body,
        grid=(n_blocks,),
        in_specs=[
            pl.BlockSpec((tb, H), lambda i: (tmap(i), 0)),
            pl.BlockSpec((1, 1, tb), lambda i: (tmap(i), 0, 0)),
            pl.BlockSpec((1, 1, tb), lambda i: (tmap(i), 0, 0)),
            pl.BlockSpec((1, 1, tb), lambda i: (tmap(i), 0, 0)),
            pl.BlockSpec((1, 1, tb), lambda i: (tmap(i), 0, 0)),
            pl.BlockSpec((POS2D, H), full),
            pl.BlockSpec((POS2D, H), full),
            pl.BlockSpec((POS2D, H), full),
            pl.BlockSpec((POS2D, H), full),
            pl.BlockSpec((1, 1), full),
            pl.BlockSpec((tb, H), lambda i: (lax.div(i, per_pos), 0)),
            pl.BlockSpec((2, H), full),
            pl.BlockSpec((H, H), full),
            pl.BlockSpec((1, H), full),
            pl.BlockSpec((H, H), full),
            pl.BlockSpec((1, H), full),
            pl.BlockSpec((1, H), full),
            pl.BlockSpec((1, H), full),
        ],
        out_specs=pl.BlockSpec((tb, H), lambda i: (tmap(i), 0)),
        out_shape=jax.ShapeDtypeStruct((n_tokens, H), jnp.float32),
        compiler_params=pltpu.CompilerParams(
            dimension_semantics=("parallel",)),
    )


def kernel(input_ids, bbox, word_emb, exp_pos_emb, x_emb, y_emb, h_emb,
           w_emb, tok_type_emb, W1, b1, W2, b2, ln_g, ln_b):
    b, s = input_ids.shape
    n = b * s
    tb = 1024
    ids = input_ids.reshape(n)
    x0 = bbox[:, :, 0].reshape(n // tb, 1, tb)
    y1 = bbox[:, :, 1].reshape(n // tb, 1, tb)
    x2 = bbox[:, :, 2].reshape(n // tb, 1, tb)
    y3 = bbox[:, :, 3].reshape(n // tb, 1, tb)

    sc = _make_sc_words(n)
    words = sc(ids, word_emb)

    absmax = jnp.maximum(
        jnp.maximum(jnp.max(jnp.abs(x_emb)), jnp.max(jnp.abs(y_emb))),
        jnp.maximum(jnp.max(jnp.abs(h_emb)), jnp.max(jnp.abs(w_emb))))
    scale = jnp.where(absmax > 0, absmax / 127.0, 1.0)
    quant = lambda t: jnp.round(t / scale).astype(jnp.int8)

    tc = _make_tc(n, s, tb)
    out = tc(words, x0, y1, x2, y3,
             quant(x_emb), quant(y_emb), quant(h_emb), quant(w_emb),
             scale.reshape(1, 1), exp_pos_emb, tok_type_emb,
             W1.astype(jnp.bfloat16), b1.reshape(1, H),
             W2.astype(jnp.bfloat16), b2.reshape(1, H),
             ln_g.reshape(1, H), ln_b.reshape(1, H))
    return out.reshape(b, s, H)
